# Initial kernel scaffold; baseline (speedup 1.0000x reference)
import jax
import jax.numpy as jnp
from jax.experimental import pallas as pl


def kernel(content_x, bert_x, edge_index, batch, content_W1, content_b1, content_W2, content_b2, bert_W1, bert_b1, bert_W2, bert_b2, lin1_W, lin1_b, lin2_W, lin2_b):
    raise NotImplementedError("write your pallas kernel here")


# speedup vs baseline: 5.5482x; 5.5482x over previous
import functools

import jax
import jax.numpy as jnp
from jax import lax
from jax.experimental import pallas as pl
from jax.experimental.pallas import tpu as pltpu
from jax.experimental.pallas import tpu_sc as plsc

N = 50000
E = 800000
NGRAPH = 128
HID = 64
NFEAT = 2 * HID

NC = 2
NS = 16
LANES = 16

EPT = 50048
NCHUNK = EPT // 128
E_PAD = NS * EPT

RNG = 12544
NPAD = 4 * RNG
TRASH = RNG
ACC_ROWS = 12800
ROWS_PER_TILE_OUT = RNG // NS

DRNG = 25088
DPAD = 2 * DRNG
DTRASH = DRNG
DACC = 25600
DOUT_PER_TILE = DRNG // NS

M_BLK = 400
GRID_M = N // M_BLK
DC_PAD = 384


_SC_MESH = dict(core_axis_name="c", subcore_axis_name="s")


@functools.partial(
    pl.kernel,
    mesh=plsc.VectorSubcoreMesh(**_SC_MESH),
    out_type=jax.ShapeDtypeStruct((DPAD,), jnp.float32),
    scratch_types=[
        pltpu.VMEM((128,), jnp.int32),
        pltpu.VMEM((128,), jnp.int32),
        pltpu.VMEM((128,), jnp.float32),
        pltpu.VMEM((1600,), jnp.float32),
        pltpu.VMEM((DOUT_PER_TILE,), jnp.float32),
        pltpu.VMEM_SHARED((DACC,), jnp.float32),
    ],
)
def _deg_sc(dst_hbm, deg_hbm, dstv, idxv, onesv, zb, dbuf, acc):
    core = lax.axis_index("c")
    sub = lax.axis_index("s")
    r0 = core * DRNG

    for j in range(8):
        onesv[pl.ds(j * 16, 16)] = jnp.ones((16,), jnp.float32)

    def zb_body(i, _):
        zb[pl.ds(i * 16, 16)] = jnp.zeros((16,), jnp.float32)
        return 0

    lax.fori_loop(0, 100, zb_body, 0)
    pltpu.sync_copy(zb, acc.at[pl.ds(sub * 1600, 1600)])
    plsc.subcore_barrier()

    def chunk(c, _):
        base = sub * EPT + c * 128
        pltpu.sync_copy(dst_hbm.at[pl.ds(base, 128)], dstv)
        for j in range(8):
            d = dstv[pl.ds(j * 16, 16)]
            ok = (d >= r0) & (d < r0 + DRNG)
            idxv[pl.ds(j * 16, 16)] = jnp.where(ok, d - r0, DTRASH)
        pltpu.sync_copy(onesv, acc.at[idxv], add=True)
        return 0

    lax.fori_loop(0, NCHUNK, chunk, 0)
    plsc.subcore_barrier()
    pltpu.sync_copy(acc.at[pl.ds(sub * DOUT_PER_TILE, DOUT_PER_TILE)], dbuf)
    pltpu.sync_copy(dbuf,
                    deg_hbm.at[pl.ds(r0 + sub * DOUT_PER_TILE,
                                     DOUT_PER_TILE)])


@functools.partial(
    pl.kernel,
    mesh=plsc.VectorSubcoreMesh(**_SC_MESH),
    out_type=jax.ShapeDtypeStruct((NPAD, NFEAT), jnp.float32),
    scratch_types=[
        pltpu.VMEM((128,), jnp.int32),
        pltpu.VMEM((128,), jnp.int32),
        pltpu.VMEM((128,), jnp.int32),
        pltpu.VMEM((128, NFEAT), jnp.float32),
        pltpu.VMEM((80, NFEAT), jnp.float32),
        pltpu.VMEM_SHARED((ACC_ROWS, NFEAT), jnp.float32),
        pltpu.SemaphoreType.DMA,
    ],
)
def _prop_sc(src_hbm, dst_hbm, h_hbm, out_hbm, srcv, dstv, idxv, rows, zbuf,
             acc, sem):
    core = lax.axis_index("c")
    sub = lax.axis_index("s")

    def zb_body(i, _):
        for j in range(NFEAT // 16):
            zbuf[i, pl.ds(j * 16, 16)] = jnp.zeros((16,), jnp.float32)
        return 0

    lax.fori_loop(0, 80, zb_body, 0)

    for p in range(2):
        r0 = (core * 2 + p) * RNG
        for i in range(10):
            pltpu.sync_copy(zbuf, acc.at[pl.ds(sub * 800 + i * 80, 80)])
        plsc.subcore_barrier()

        def chunk(c, _):
            base = sub * EPT + c * 128
            pltpu.sync_copy(src_hbm.at[pl.ds(base, 128)], srcv)
            pltpu.sync_copy(dst_hbm.at[pl.ds(base, 128)], dstv)
            for j in range(8):
                d = dstv[pl.ds(j * 16, 16)]
                ok = (d >= r0) & (d < r0 + RNG)
                idxv[pl.ds(j * 16, 16)] = jnp.where(ok, d - r0, TRASH)
            pltpu.async_copy(h_hbm.at[srcv], rows, sem).wait()
            pltpu.sync_copy(rows, acc.at[idxv], add=True)
            return 0

        lax.fori_loop(0, NCHUNK, chunk, 0)
        plsc.subcore_barrier()
        obase = sub * ROWS_PER_TILE_OUT
        for k in range(6):
            pltpu.sync_copy(acc.at[pl.ds(obase + k * 128, 128)], rows)
            pltpu.sync_copy(rows,
                            out_hbm.at[pl.ds(r0 + obase + k * 128, 128)])
        tail = ROWS_PER_TILE_OUT - 6 * 128
        pltpu.sync_copy(acc.at[pl.ds(obase + 768, tail)],
                        rows.at[pl.ds(0, tail)])
        pltpu.sync_copy(rows.at[pl.ds(0, tail)],
                        out_hbm.at[pl.ds(r0 + obase + 768, tail)])
        plsc.subcore_barrier()


def _mm1_body(xc_ref, xb_ref, wc_ref, wb_ref, deg_ref, out_ref):
    dinv = lax.rsqrt(deg_ref[...] + 1.0)
    hc = jnp.dot(xc_ref[...], wc_ref[...], preferred_element_type=jnp.float32)
    hb = jnp.dot(xb_ref[...], wb_ref[...], preferred_element_type=jnp.float32)
    out_ref[...] = jnp.concatenate([hc, hb], axis=1) * dinv


def _mm2_body(s_ref, h_ref, deg_ref, wc_ref, wb_ref, bc_ref, bb_ref, out_ref):
    dinv = lax.rsqrt(deg_ref[...] + 1.0)
    o = (s_ref[...] + h_ref[...]) * dinv
    oc = o[:, :HID] + bc_ref[...]
    ob = o[:, HID:] + bb_ref[...]
    h2c = jnp.dot(oc, wc_ref[...], preferred_element_type=jnp.float32)
    h2b = jnp.dot(ob, wb_ref[...], preferred_element_type=jnp.float32)
    out_ref[...] = jnp.concatenate([h2c, h2b], axis=1) * dinv


def _fin_body(s_ref, h_ref, deg_ref, bc_ref, bb_ref, batch_ref,
              w1_ref, b1_ref, w2_ref, b2_ref, out_ref, sums, cnts):
    pid = pl.program_id(0)

    @pl.when(pid == 0)
    def _():
        sums[...] = jnp.zeros_like(sums)
        cnts[...] = jnp.zeros_like(cnts)

    dinv = lax.rsqrt(deg_ref[...] + 1.0)
    o = (s_ref[...] + h_ref[...]) * dinv
    o = o + jnp.concatenate([bc_ref[...], bb_ref[...]], axis=1)
    b = batch_ref[0, 0, :]
    oh = (b[:, None] == lax.broadcasted_iota(jnp.int32, (M_BLK, NGRAPH), 1)
          ).astype(jnp.float32)
    sums[...] += lax.dot_general(oh, o, (((0,), (0,)), ((), ())),
                                 preferred_element_type=jnp.float32)
    cnts[...] += lax.dot_general(oh, jnp.ones((M_BLK, NFEAT), jnp.float32),
                                 (((0,), (0,)), ((), ())),
                                 preferred_element_type=jnp.float32)

    @pl.when(pid == GRID_M - 1)
    def _():
        pooled = sums[...] / jnp.maximum(cnts[...], 1.0)
        hmlp = jax.nn.relu(
            jnp.dot(pooled, w1_ref[...], preferred_element_type=jnp.float32)
            + b1_ref[...])
        lg = jnp.dot(hmlp, w2_ref[...],
                     preferred_element_type=jnp.float32) + b2_ref[...]
        m = jnp.max(lg, axis=1, keepdims=True)
        z = lg - m
        out_ref[...] = z - jnp.log(jnp.sum(jnp.exp(z), axis=1, keepdims=True))


def _row_spec(w):
    return pl.BlockSpec((M_BLK, w), lambda i: (i, 0))


def _full_spec(r, c):
    return pl.BlockSpec((r, c), lambda i: (0, 0))


_mm1 = pl.pallas_call(
    _mm1_body,
    grid=(GRID_M,),
    in_specs=[
        _row_spec(DC_PAD), _row_spec(768),
        _full_spec(DC_PAD, HID), _full_spec(768, HID), _row_spec(1),
    ],
    out_specs=_row_spec(NFEAT),
    out_shape=jax.ShapeDtypeStruct((N, NFEAT), jnp.float32),
)

_mm2 = pl.pallas_call(
    _mm2_body,
    grid=(GRID_M,),
    in_specs=[
        _row_spec(NFEAT), _row_spec(NFEAT), _row_spec(1),
        _full_spec(HID, HID), _full_spec(HID, HID),
        _full_spec(1, HID), _full_spec(1, HID),
    ],
    out_specs=_row_spec(NFEAT),
    out_shape=jax.ShapeDtypeStruct((N, NFEAT), jnp.float32),
)

_fin = pl.pallas_call(
    _fin_body,
    grid=(GRID_M,),
    in_specs=[
        _row_spec(NFEAT), _row_spec(NFEAT), _row_spec(1),
        _full_spec(1, HID), _full_spec(1, HID),
        pl.BlockSpec((1, 1, M_BLK), lambda i: (i, 0, 0)),
        _full_spec(NFEAT, HID), _full_spec(1, HID),
        _full_spec(HID, 8), _full_spec(1, 8),
    ],
    out_specs=_full_spec(NGRAPH, 8),
    out_shape=jax.ShapeDtypeStruct((NGRAPH, 8), jnp.float32),
    scratch_shapes=[
        pltpu.VMEM((NGRAPH, NFEAT), jnp.float32),
        pltpu.VMEM((NGRAPH, NFEAT), jnp.float32),
    ],
)


def kernel(content_x, bert_x, edge_index, batch,
           content_W1, content_b1, content_W2, content_b2,
           bert_W1, bert_b1, bert_W2, bert_b2,
           lin1_W, lin1_b, lin2_W, lin2_b):
    xc = jnp.pad(content_x, ((0, 0), (0, DC_PAD - content_x.shape[1])))
    wc1 = jnp.pad(content_W1, ((0, DC_PAD - content_W1.shape[0]), (0, 0)))

    esrc = jnp.pad(edge_index[0], (0, E_PAD - E))
    edst = jnp.pad(edge_index[1], (0, E_PAD - E),
                   constant_values=jnp.int32(1 << 28))

    deg = _deg_sc(edst)[:N].reshape(N, 1)

    h1p = _mm1(xc, bert_x, wc1, bert_W1, deg)
    s1 = _prop_sc(esrc, edst, h1p)[:N]
    h2p = _mm2(s1, h1p, deg, content_W2, bert_W2,
               content_b1.reshape(1, HID), bert_b1.reshape(1, HID))
    s2 = _prop_sc(esrc, edst, h2p)[:N]
    return _fin(s2, h2p, deg,
                content_b2.reshape(1, HID), bert_b2.reshape(1, HID),
                batch.reshape(GRID_M, 1, M_BLK),
                lin1_W, lin1_b.reshape(1, HID),
                lin2_W, lin2_b.reshape(1, 8))


# speedup vs baseline: 5.8344x; 1.0516x over previous
import functools

import jax
import jax.numpy as jnp
from jax import lax
from jax.experimental import pallas as pl
from jax.experimental.pallas import tpu as pltpu
from jax.experimental.pallas import tpu_sc as plsc

N = 50000
E = 800000
NGRAPH = 128
HID = 64
NFEAT = 2 * HID

NC = 2
NS = 16
LANES = 16

EPT = 50048
NCHUNK = EPT // 128
E_PAD = NS * EPT

RNG = 12544
NPAD = 4 * RNG
TRASH = RNG
ACC_ROWS = 12800
ROWS_PER_TILE_OUT = RNG // NS

DRNG = 25088
DPAD = 2 * DRNG
DTRASH = DRNG
DACC = 25600
DOUT_PER_TILE = DRNG // NS

M_BLK = 400
GRID_M = N // M_BLK
DC_PAD = 384


_SC_MESH = dict(core_axis_name="c", subcore_axis_name="s")


@functools.partial(
    pl.kernel,
    mesh=plsc.VectorSubcoreMesh(**_SC_MESH),
    out_type=jax.ShapeDtypeStruct((DPAD,), jnp.float32),
    scratch_types=[
        pltpu.VMEM((128,), jnp.int32),
        pltpu.VMEM((128,), jnp.int32),
        pltpu.VMEM((128,), jnp.float32),
        pltpu.VMEM((1600,), jnp.float32),
        pltpu.VMEM((DOUT_PER_TILE,), jnp.float32),
        pltpu.VMEM_SHARED((DACC,), jnp.float32),
    ],
)
def _deg_sc(dst_hbm, deg_hbm, dstv, idxv, onesv, zb, dbuf, acc):
    core = lax.axis_index("c")
    sub = lax.axis_index("s")
    r0 = core * DRNG

    for j in range(8):
        onesv[pl.ds(j * 16, 16)] = jnp.ones((16,), jnp.float32)

    def zb_body(i, _):
        zb[pl.ds(i * 16, 16)] = jnp.zeros((16,), jnp.float32)
        return 0

    lax.fori_loop(0, 100, zb_body, 0)
    pltpu.sync_copy(zb, acc.at[pl.ds(sub * 1600, 1600)])
    plsc.subcore_barrier()

    def chunk(c, _):
        base = sub * EPT + c * 128
        pltpu.sync_copy(dst_hbm.at[pl.ds(base, 128)], dstv)
        for j in range(8):
            d = dstv[pl.ds(j * 16, 16)]
            ok = (d >= r0) & (d < r0 + DRNG)
            trash = DTRASH + j * 16 + lax.iota(jnp.int32, 16)
            idxv[pl.ds(j * 16, 16)] = jnp.where(ok, d - r0, trash)
        pltpu.sync_copy(onesv, acc.at[idxv], add=True)
        return 0

    lax.fori_loop(0, NCHUNK, chunk, 0)
    plsc.subcore_barrier()
    pltpu.sync_copy(acc.at[pl.ds(sub * DOUT_PER_TILE, DOUT_PER_TILE)], dbuf)
    pltpu.sync_copy(dbuf,
                    deg_hbm.at[pl.ds(r0 + sub * DOUT_PER_TILE,
                                     DOUT_PER_TILE)])


@functools.partial(
    pl.kernel,
    mesh=plsc.VectorSubcoreMesh(**_SC_MESH),
    out_type=jax.ShapeDtypeStruct((NPAD, NFEAT), jnp.float32),
    scratch_types=[
        pltpu.VMEM((128,), jnp.int32),
        pltpu.VMEM((128,), jnp.int32),
        pltpu.VMEM((128,), jnp.int32),
        pltpu.VMEM((128, NFEAT), jnp.float32),
        pltpu.VMEM((80, NFEAT), jnp.float32),
        pltpu.VMEM_SHARED((ACC_ROWS, NFEAT), jnp.float32),
        pltpu.SemaphoreType.DMA,
    ],
)
def _prop_sc(src_hbm, dst_hbm, h_hbm, out_hbm, srcv, dstv, idxv, rows, zbuf,
             acc, sem):
    core = lax.axis_index("c")
    sub = lax.axis_index("s")

    def zb_body(i, _):
        for j in range(NFEAT // 16):
            zbuf[i, pl.ds(j * 16, 16)] = jnp.zeros((16,), jnp.float32)
        return 0

    lax.fori_loop(0, 80, zb_body, 0)

    for p in range(2):
        r0 = (core * 2 + p) * RNG
        for i in range(10):
            pltpu.sync_copy(zbuf, acc.at[pl.ds(sub * 800 + i * 80, 80)])
        plsc.subcore_barrier()

        def chunk(c, _):
            base = sub * EPT + c * 128
            pltpu.sync_copy(src_hbm.at[pl.ds(base, 128)], srcv)
            pltpu.sync_copy(dst_hbm.at[pl.ds(base, 128)], dstv)
            for j in range(8):
                d = dstv[pl.ds(j * 16, 16)]
                ok = (d >= r0) & (d < r0 + RNG)
                trash = TRASH + j * 16 + lax.iota(jnp.int32, 16)
                idxv[pl.ds(j * 16, 16)] = jnp.where(ok, d - r0, trash)
            pltpu.async_copy(h_hbm.at[srcv], rows, sem).wait()
            pltpu.sync_copy(rows, acc.at[idxv], add=True)
            return 0

        lax.fori_loop(0, NCHUNK, chunk, 0)
        plsc.subcore_barrier()
        obase = sub * ROWS_PER_TILE_OUT
        for k in range(6):
            pltpu.sync_copy(acc.at[pl.ds(obase + k * 128, 128)], rows)
            pltpu.sync_copy(rows,
                            out_hbm.at[pl.ds(r0 + obase + k * 128, 128)])
        tail = ROWS_PER_TILE_OUT - 6 * 128
        pltpu.sync_copy(acc.at[pl.ds(obase + 768, tail)],
                        rows.at[pl.ds(0, tail)])
        pltpu.sync_copy(rows.at[pl.ds(0, tail)],
                        out_hbm.at[pl.ds(r0 + obase + 768, tail)])
        plsc.subcore_barrier()


def _mm1_body(xc_ref, xb_ref, wc_ref, wb_ref, deg_ref, out_ref):
    dinv = lax.rsqrt(deg_ref[...] + 1.0)
    hc = jnp.dot(xc_ref[...], wc_ref[...], preferred_element_type=jnp.float32)
    hb = jnp.dot(xb_ref[...], wb_ref[...], preferred_element_type=jnp.float32)
    out_ref[...] = jnp.concatenate([hc, hb], axis=1) * dinv


def _mm2_body(s_ref, h_ref, deg_ref, wc_ref, wb_ref, bc_ref, bb_ref, out_ref):
    dinv = lax.rsqrt(deg_ref[...] + 1.0)
    o = (s_ref[...] + h_ref[...]) * dinv
    oc = o[:, :HID] + bc_ref[...]
    ob = o[:, HID:] + bb_ref[...]
    h2c = jnp.dot(oc, wc_ref[...], preferred_element_type=jnp.float32)
    h2b = jnp.dot(ob, wb_ref[...], preferred_element_type=jnp.float32)
    out_ref[...] = jnp.concatenate([h2c, h2b], axis=1) * dinv


def _fin_body(s_ref, h_ref, deg_ref, bc_ref, bb_ref, batch_ref,
              w1_ref, b1_ref, w2_ref, b2_ref, out_ref, sums, cnts):
    pid = pl.program_id(0)

    @pl.when(pid == 0)
    def _():
        sums[...] = jnp.zeros_like(sums)
        cnts[...] = jnp.zeros_like(cnts)

    dinv = lax.rsqrt(deg_ref[...] + 1.0)
    o = (s_ref[...] + h_ref[...]) * dinv
    o = o + jnp.concatenate([bc_ref[...], bb_ref[...]], axis=1)
    b = batch_ref[0, 0, :]
    oh = (b[:, None] == lax.broadcasted_iota(jnp.int32, (M_BLK, NGRAPH), 1)
          ).astype(jnp.float32)
    sums[...] += lax.dot_general(oh, o, (((0,), (0,)), ((), ())),
                                 preferred_element_type=jnp.float32)
    cnts[...] += lax.dot_general(oh, jnp.ones((M_BLK, NFEAT), jnp.float32),
                                 (((0,), (0,)), ((), ())),
                                 preferred_element_type=jnp.float32)

    @pl.when(pid == GRID_M - 1)
    def _():
        pooled = sums[...] / jnp.maximum(cnts[...], 1.0)
        hmlp = jax.nn.relu(
            jnp.dot(pooled, w1_ref[...], preferred_element_type=jnp.float32)
            + b1_ref[...])
        lg = jnp.dot(hmlp, w2_ref[...],
                     preferred_element_type=jnp.float32) + b2_ref[...]
        m = jnp.max(lg, axis=1, keepdims=True)
        z = lg - m
        out_ref[...] = z - jnp.log(jnp.sum(jnp.exp(z), axis=1, keepdims=True))


def _row_spec(w):
    return pl.BlockSpec((M_BLK, w), lambda i: (i, 0))


def _full_spec(r, c):
    return pl.BlockSpec((r, c), lambda i: (0, 0))


_mm1 = pl.pallas_call(
    _mm1_body,
    grid=(GRID_M,),
    in_specs=[
        _row_spec(DC_PAD), _row_spec(768),
        _full_spec(DC_PAD, HID), _full_spec(768, HID), _row_spec(1),
    ],
    out_specs=_row_spec(NFEAT),
    out_shape=jax.ShapeDtypeStruct((N, NFEAT), jnp.float32),
)

_mm2 = pl.pallas_call(
    _mm2_body,
    grid=(GRID_M,),
    in_specs=[
        _row_spec(NFEAT), _row_spec(NFEAT), _row_spec(1),
        _full_spec(HID, HID), _full_spec(HID, HID),
        _full_spec(1, HID), _full_spec(1, HID),
    ],
    out_specs=_row_spec(NFEAT),
    out_shape=jax.ShapeDtypeStruct((N, NFEAT), jnp.float32),
)

_fin = pl.pallas_call(
    _fin_body,
    grid=(GRID_M,),
    in_specs=[
        _row_spec(NFEAT), _row_spec(NFEAT), _row_spec(1),
        _full_spec(1, HID), _full_spec(1, HID),
        pl.BlockSpec((1, 1, M_BLK), lambda i: (i, 0, 0)),
        _full_spec(NFEAT, HID), _full_spec(1, HID),
        _full_spec(HID, 8), _full_spec(1, 8),
    ],
    out_specs=_full_spec(NGRAPH, 8),
    out_shape=jax.ShapeDtypeStruct((NGRAPH, 8), jnp.float32),
    scratch_shapes=[
        pltpu.VMEM((NGRAPH, NFEAT), jnp.float32),
        pltpu.VMEM((NGRAPH, NFEAT), jnp.float32),
    ],
)


def kernel(content_x, bert_x, edge_index, batch,
           content_W1, content_b1, content_W2, content_b2,
           bert_W1, bert_b1, bert_W2, bert_b2,
           lin1_W, lin1_b, lin2_W, lin2_b):
    xc = jnp.pad(content_x, ((0, 0), (0, DC_PAD - content_x.shape[1])))
    wc1 = jnp.pad(content_W1, ((0, DC_PAD - content_W1.shape[0]), (0, 0)))

    esrc = jnp.pad(edge_index[0], (0, E_PAD - E))
    edst = jnp.pad(edge_index[1], (0, E_PAD - E),
                   constant_values=jnp.int32(1 << 28))

    deg = _deg_sc(edst)[:N].reshape(N, 1)

    h1p = _mm1(xc, bert_x, wc1, bert_W1, deg)
    s1 = _prop_sc(esrc, edst, h1p)[:N]
    h2p = _mm2(s1, h1p, deg, content_W2, bert_W2,
               content_b1.reshape(1, HID), bert_b1.reshape(1, HID))
    s2 = _prop_sc(esrc, edst, h2p)[:N]
    return _fin(s2, h2p, deg,
                content_b2.reshape(1, HID), bert_b2.reshape(1, HID),
                batch.reshape(GRID_M, 1, M_BLK),
                lin1_W, lin1_b.reshape(1, HID),
                lin2_W, lin2_b.reshape(1, 8))


# speedup vs baseline: 6.6141x; 1.1336x over previous
import functools

import jax
import jax.numpy as jnp
from jax import lax
from jax.experimental import pallas as pl
from jax.experimental.pallas import tpu as pltpu
from jax.experimental.pallas import tpu_sc as plsc

N = 50000
E = 800000
NGRAPH = 128
HID = 64
NFEAT = 2 * HID

NC = 2
NS = 16
LANES = 16

EPT = 50048
NCHUNK = EPT // 128
E_PAD = NS * EPT

RNG = 12544
NPAD = 4 * RNG
TRASH = RNG
ACC_ROWS = 12800
ROWS_PER_TILE_OUT = RNG // NS

DRNG = 25088
DPAD = 2 * DRNG
DTRASH = DRNG
DACC = 25600
DOUT_PER_TILE = DRNG // NS

M_BLK = 400
GRID_M = N // M_BLK
DC_PAD = 384


_SC_MESH = dict(core_axis_name="c", subcore_axis_name="s")


@functools.partial(
    pl.kernel,
    mesh=plsc.VectorSubcoreMesh(**_SC_MESH),
    out_type=jax.ShapeDtypeStruct((DPAD,), jnp.float32),
    scratch_types=[
        pltpu.VMEM((128,), jnp.int32),
        pltpu.VMEM((128,), jnp.int32),
        pltpu.VMEM((128,), jnp.float32),
        pltpu.VMEM((1600,), jnp.float32),
        pltpu.VMEM((DOUT_PER_TILE,), jnp.float32),
        pltpu.VMEM_SHARED((DACC,), jnp.float32),
    ],
)
def _deg_sc(dst_hbm, deg_hbm, dstv, idxv, onesv, zb, dbuf, acc):
    core = lax.axis_index("c")
    sub = lax.axis_index("s")
    r0 = core * DRNG

    for j in range(8):
        onesv[pl.ds(j * 16, 16)] = jnp.ones((16,), jnp.float32)

    def zb_body(i, _):
        zb[pl.ds(i * 16, 16)] = jnp.zeros((16,), jnp.float32)
        return 0

    lax.fori_loop(0, 100, zb_body, 0)
    pltpu.sync_copy(zb, acc.at[pl.ds(sub * 1600, 1600)])
    plsc.subcore_barrier()

    def chunk(c, _):
        base = sub * EPT + c * 128
        pltpu.sync_copy(dst_hbm.at[pl.ds(base, 128)], dstv)
        for j in range(8):
            d = dstv[pl.ds(j * 16, 16)]
            ok = (d >= r0) & (d < r0 + DRNG)
            trash = DTRASH + j * 16 + lax.iota(jnp.int32, 16)
            idxv[pl.ds(j * 16, 16)] = jnp.where(ok, d - r0, trash)
        pltpu.sync_copy(onesv, acc.at[idxv], add=True)
        return 0

    lax.fori_loop(0, NCHUNK, chunk, 0)
    plsc.subcore_barrier()
    pltpu.sync_copy(acc.at[pl.ds(sub * DOUT_PER_TILE, DOUT_PER_TILE)], dbuf)
    pltpu.sync_copy(dbuf,
                    deg_hbm.at[pl.ds(r0 + sub * DOUT_PER_TILE,
                                     DOUT_PER_TILE)])


@functools.partial(
    pl.kernel,
    mesh=plsc.VectorSubcoreMesh(**_SC_MESH),
    out_type=jax.ShapeDtypeStruct((NPAD, NFEAT), jnp.float32),
    scratch_types=[
        pltpu.VMEM((64,), jnp.int32),
        pltpu.VMEM((64,), jnp.int32),
        pltpu.VMEM((64,), jnp.int32),
        pltpu.VMEM((64,), jnp.int32),
        pltpu.VMEM((64,), jnp.int32),
        pltpu.VMEM((64, NFEAT), jnp.float32),
        pltpu.VMEM((64, NFEAT), jnp.float32),
        pltpu.VMEM((80, NFEAT), jnp.float32),
        pltpu.VMEM_SHARED((ACC_ROWS, NFEAT), jnp.float32),
        pltpu.SemaphoreType.DMA,
        pltpu.SemaphoreType.DMA,
    ],
)
def _prop_sc(src_hbm, dst_hbm, h_hbm, out_hbm, srcA, srcB, dstv, idxA, idxB,
             rowsA, rowsB, zbuf, acc, semA, semB):
    core = lax.axis_index("c")
    sub = lax.axis_index("s")
    ebase = sub * EPT
    nhalf = EPT // 128

    def zb_body(i, _):
        for j in range(NFEAT // 16):
            zbuf[i, pl.ds(j * 16, 16)] = jnp.zeros((16,), jnp.float32)
        return 0

    lax.fori_loop(0, 80, zb_body, 0)

    def _stage(c, srcv, idxv, r0):
        base = ebase + c * 64
        pltpu.sync_copy(src_hbm.at[pl.ds(base, 64)], srcv)
        pltpu.sync_copy(dst_hbm.at[pl.ds(base, 64)], dstv)
        for j in range(4):
            d = dstv[pl.ds(j * 16, 16)]
            ok = (d >= r0) & (d < r0 + RNG)
            trash = TRASH + j * 16 + lax.iota(jnp.int32, 16)
            idxv[pl.ds(j * 16, 16)] = jnp.where(ok, d - r0, trash)

    for p in range(2):
        r0 = (core * 2 + p) * RNG
        for i in range(10):
            pltpu.sync_copy(zbuf, acc.at[pl.ds(sub * 800 + i * 80, 80)])
        plsc.subcore_barrier()

        _stage(0, srcA, idxA, r0)
        pltpu.async_copy(h_hbm.at[srcA], rowsA, semA)

        def dchunk(i, _):
            _stage(2 * i + 1, srcB, idxB, r0)
            pltpu.async_copy(h_hbm.at[srcB], rowsB, semB)
            pltpu.make_async_copy(h_hbm.at[srcA], rowsA, semA).wait()
            pltpu.sync_copy(rowsA, acc.at[idxA], add=True)

            @pl.when(i < nhalf - 1)
            def _():
                _stage(2 * i + 2, srcA, idxA, r0)
                pltpu.async_copy(h_hbm.at[srcA], rowsA, semA)

            pltpu.make_async_copy(h_hbm.at[srcB], rowsB, semB).wait()
            pltpu.sync_copy(rowsB, acc.at[idxB], add=True)
            return 0

        lax.fori_loop(0, nhalf, dchunk, 0)
        plsc.subcore_barrier()
        obase = sub * ROWS_PER_TILE_OUT
        for k in range(12):
            pltpu.sync_copy(acc.at[pl.ds(obase + k * 64, 64)], rowsA)
            pltpu.sync_copy(rowsA,
                            out_hbm.at[pl.ds(r0 + obase + k * 64, 64)])
        tail = ROWS_PER_TILE_OUT - 12 * 64
        pltpu.sync_copy(acc.at[pl.ds(obase + 768, tail)],
                        rowsA.at[pl.ds(0, tail)])
        pltpu.sync_copy(rowsA.at[pl.ds(0, tail)],
                        out_hbm.at[pl.ds(r0 + obase + 768, tail)])
        plsc.subcore_barrier()


def _mm1_body(xc_ref, xb_ref, wc_ref, wb_ref, deg_ref, out_ref):
    dinv = lax.rsqrt(deg_ref[...] + 1.0)
    hc = jnp.dot(xc_ref[...], wc_ref[...], preferred_element_type=jnp.float32)
    hb = jnp.dot(xb_ref[...], wb_ref[...], preferred_element_type=jnp.float32)
    out_ref[...] = jnp.concatenate([hc, hb], axis=1) * dinv


def _mm2_body(s_ref, h_ref, deg_ref, wc_ref, wb_ref, bc_ref, bb_ref, out_ref):
    dinv = lax.rsqrt(deg_ref[...] + 1.0)
    o = (s_ref[...] + h_ref[...]) * dinv
    oc = o[:, :HID] + bc_ref[...]
    ob = o[:, HID:] + bb_ref[...]
    h2c = jnp.dot(oc, wc_ref[...], preferred_element_type=jnp.float32)
    h2b = jnp.dot(ob, wb_ref[...], preferred_element_type=jnp.float32)
    out_ref[...] = jnp.concatenate([h2c, h2b], axis=1) * dinv


def _fin_body(s_ref, h_ref, deg_ref, bc_ref, bb_ref, batch_ref,
              w1_ref, b1_ref, w2_ref, b2_ref, out_ref, sums, cnts):
    pid = pl.program_id(0)

    @pl.when(pid == 0)
    def _():
        sums[...] = jnp.zeros_like(sums)
        cnts[...] = jnp.zeros_like(cnts)

    dinv = lax.rsqrt(deg_ref[...] + 1.0)
    o = (s_ref[...] + h_ref[...]) * dinv
    o = o + jnp.concatenate([bc_ref[...], bb_ref[...]], axis=1)
    b = batch_ref[0, 0, :]
    oh = (b[:, None] == lax.broadcasted_iota(jnp.int32, (M_BLK, NGRAPH), 1)
          ).astype(jnp.float32)
    sums[...] += lax.dot_general(oh, o, (((0,), (0,)), ((), ())),
                                 preferred_element_type=jnp.float32)
    cnts[...] += lax.dot_general(oh, jnp.ones((M_BLK, NFEAT), jnp.float32),
                                 (((0,), (0,)), ((), ())),
                                 preferred_element_type=jnp.float32)

    @pl.when(pid == GRID_M - 1)
    def _():
        pooled = sums[...] / jnp.maximum(cnts[...], 1.0)
        hmlp = jax.nn.relu(
            jnp.dot(pooled, w1_ref[...], preferred_element_type=jnp.float32)
            + b1_ref[...])
        lg = jnp.dot(hmlp, w2_ref[...],
                     preferred_element_type=jnp.float32) + b2_ref[...]
        m = jnp.max(lg, axis=1, keepdims=True)
        z = lg - m
        out_ref[...] = z - jnp.log(jnp.sum(jnp.exp(z), axis=1, keepdims=True))


def _row_spec(w):
    return pl.BlockSpec((M_BLK, w), lambda i: (i, 0))


def _full_spec(r, c):
    return pl.BlockSpec((r, c), lambda i: (0, 0))


_mm1 = pl.pallas_call(
    _mm1_body,
    grid=(GRID_M,),
    in_specs=[
        _row_spec(DC_PAD), _row_spec(768),
        _full_spec(DC_PAD, HID), _full_spec(768, HID), _row_spec(1),
    ],
    out_specs=_row_spec(NFEAT),
    out_shape=jax.ShapeDtypeStruct((N, NFEAT), jnp.float32),
)

_mm2 = pl.pallas_call(
    _mm2_body,
    grid=(GRID_M,),
    in_specs=[
        _row_spec(NFEAT), _row_spec(NFEAT), _row_spec(1),
        _full_spec(HID, HID), _full_spec(HID, HID),
        _full_spec(1, HID), _full_spec(1, HID),
    ],
    out_specs=_row_spec(NFEAT),
    out_shape=jax.ShapeDtypeStruct((N, NFEAT), jnp.float32),
)

_fin = pl.pallas_call(
    _fin_body,
    grid=(GRID_M,),
    in_specs=[
        _row_spec(NFEAT), _row_spec(NFEAT), _row_spec(1),
        _full_spec(1, HID), _full_spec(1, HID),
        pl.BlockSpec((1, 1, M_BLK), lambda i: (i, 0, 0)),
        _full_spec(NFEAT, HID), _full_spec(1, HID),
        _full_spec(HID, 8), _full_spec(1, 8),
    ],
    out_specs=_full_spec(NGRAPH, 8),
    out_shape=jax.ShapeDtypeStruct((NGRAPH, 8), jnp.float32),
    scratch_shapes=[
        pltpu.VMEM((NGRAPH, NFEAT), jnp.float32),
        pltpu.VMEM((NGRAPH, NFEAT), jnp.float32),
    ],
)


def kernel(content_x, bert_x, edge_index, batch,
           content_W1, content_b1, content_W2, content_b2,
           bert_W1, bert_b1, bert_W2, bert_b2,
           lin1_W, lin1_b, lin2_W, lin2_b):
    xc = jnp.pad(content_x, ((0, 0), (0, DC_PAD - content_x.shape[1])))
    wc1 = jnp.pad(content_W1, ((0, DC_PAD - content_W1.shape[0]), (0, 0)))

    esrc = jnp.pad(edge_index[0], (0, E_PAD - E))
    edst = jnp.pad(edge_index[1], (0, E_PAD - E),
                   constant_values=jnp.int32(1 << 28))

    deg = _deg_sc(edst)[:N].reshape(N, 1)

    h1p = _mm1(xc, bert_x, wc1, bert_W1, deg)
    s1 = _prop_sc(esrc, edst, h1p)[:N]
    h2p = _mm2(s1, h1p, deg, content_W2, bert_W2,
               content_b1.reshape(1, HID), bert_b1.reshape(1, HID))
    s2 = _prop_sc(esrc, edst, h2p)[:N]
    return _fin(s2, h2p, deg,
                content_b2.reshape(1, HID), bert_b2.reshape(1, HID),
                batch.reshape(GRID_M, 1, M_BLK),
                lin1_W, lin1_b.reshape(1, HID),
                lin2_W, lin2_b.reshape(1, 8))
